# baseline (speedup 1.0000x reference)
import jax
import jax.numpy as jnp
from jax.experimental import pallas as pl

_BLK = 128


def _fused_vo_body(x_ref, wv_ref, bv_ref, wo_ref, bo_ref, y_ref):
    i = pl.program_id(0)
    v = jnp.dot(x_ref[...], wv_ref[...],
                preferred_element_type=jnp.float32) + bv_ref[...]
    contrib = jnp.dot(v, wo_ref[...], preferred_element_type=jnp.float32)

    @pl.when(i == 0)
    def _init():
        y_ref[...] = contrib + bo_ref[...]

    @pl.when(i > 0)
    def _acc():
        y_ref[...] += contrib


def kernel(x, reset_mask, Wq, bq, Wk, bk, Wv, bv, Wo, bo):
    del reset_mask, Wq, bq, Wk, bk
    bs, d = x.shape
    d_wm = Wv.shape[1]
    grid = (d_wm // _BLK,)
    return pl.pallas_call(
        _fused_vo_body,
        grid=grid,
        in_specs=[
            pl.BlockSpec((bs, d), lambda i: (0, 0)),
            pl.BlockSpec((d, _BLK), lambda i: (0, i)),
            pl.BlockSpec((1, _BLK), lambda i: (0, i)),
            pl.BlockSpec((_BLK, d), lambda i: (i, 0)),
            pl.BlockSpec((1, d), lambda i: (0, 0)),
        ],
        out_specs=pl.BlockSpec((bs, d), lambda i: (0, 0)),
        out_shape=jax.ShapeDtypeStruct((bs, d), jnp.float32),
    )(x, Wv, bv.reshape(1, -1), Wo, bo.reshape(1, -1))


# speedup vs baseline: 1.5223x; 1.5223x over previous
import jax
import jax.numpy as jnp
from jax.experimental import pallas as pl
from jax.experimental.pallas import tpu as pltpu

_NSPLIT = 2


def _fused_vo_body(x_ref, wv_ref, bv_ref, wo_ref, bo_ref, y_ref):
    v = jnp.dot(x_ref[...], wv_ref[...],
                preferred_element_type=jnp.float32) + bv_ref[...]
    y_ref[...] = jnp.dot(v, wo_ref[...],
                         preferred_element_type=jnp.float32) + bo_ref[...]


def kernel(x, reset_mask, Wq, bq, Wk, bk, Wv, bv, Wo, bo):
    del reset_mask, Wq, bq, Wk, bk
    bs, d = x.shape
    d_wm = Wv.shape[1]
    nb = d // _NSPLIT
    return pl.pallas_call(
        _fused_vo_body,
        grid=(_NSPLIT,),
        in_specs=[
            pl.BlockSpec((bs, d), lambda i: (0, 0)),
            pl.BlockSpec((d, d_wm), lambda i: (0, 0)),
            pl.BlockSpec((1, d_wm), lambda i: (0, 0)),
            pl.BlockSpec((d_wm, nb), lambda i: (0, i)),
            pl.BlockSpec((1, nb), lambda i: (0, i)),
        ],
        out_specs=pl.BlockSpec((bs, nb), lambda i: (0, i)),
        out_shape=jax.ShapeDtypeStruct((bs, d), jnp.float32),
        compiler_params=pltpu.CompilerParams(
            dimension_semantics=("parallel",),
        ),
    )(x, Wv, bv.reshape(1, -1), Wo, bo.reshape(1, -1))
